# baseline (speedup 1.0000x reference)
import functools

import jax
import jax.numpy as jnp
from jax import lax
from jax.experimental import pallas as pl
from jax.experimental.pallas import tpu as pltpu
from jax.experimental.pallas import tpu_sc as plsc

_BUFFER_SIZE = 4194304
_TAIL = 1536
_BUF_LEN = _BUFFER_SIZE + _TAIL
_NC = 2
_PER_C = _BUFFER_SIZE // _NC
_CH = 262144
_NCH = _PER_C // _CH
_NBUF = 4

_MESH = plsc.ScalarSubcoreMesh(axis_name="c", num_cores=_NC)


def _in_copy(x_hbm, stage, in_sems, base, c):
    return pltpu.make_async_copy(
        x_hbm.at[pl.ds(0, 1), pl.ds(base + c * _CH, _CH)],
        stage.at[pl.ds(c % _NBUF, 1), :],
        in_sems.at[c % _NBUF],
    )


def _out_copy(out_hbm, stage, out_sems, base, c):
    return pltpu.make_async_copy(
        stage.at[pl.ds(c % _NBUF, 1), :],
        out_hbm.at[pl.ds(0, 1), pl.ds(_TAIL + base + c * _CH, _CH)],
        out_sems.at[c % _NBUF],
    )


@functools.partial(
    pl.kernel,
    out_type=jax.ShapeDtypeStruct((1, _BUF_LEN), jnp.float32),
    mesh=_MESH,
    scratch_types=[
        pltpu.VMEM_SHARED((_NBUF, _CH), jnp.float32),
        pltpu.VMEM_SHARED((1, _TAIL), jnp.float32),
        pltpu.SemaphoreType.DMA((_NBUF,)),
        pltpu.SemaphoreType.DMA((_NBUF,)),
        pltpu.SemaphoreType.DMA,
        pltpu.SemaphoreType.DMA,
    ],
)
def _sc_concat(x_hbm, buf_hbm, out_hbm, stage, tstage, in_sems, out_sems,
               tin_sem, tout_sem):
    cid = lax.axis_index("c")
    base = cid * _PER_C

    for b in range(_NBUF):
        _in_copy(x_hbm, stage, in_sems, base, b).start()

    @pl.when(cid == 0)
    def _():
        t_in = pltpu.make_async_copy(
            buf_hbm.at[pl.ds(0, 1), pl.ds(_BUFFER_SIZE, _TAIL)], tstage,
            tin_sem)
        t_in.start()
        t_in.wait()
        pltpu.make_async_copy(
            tstage, out_hbm.at[pl.ds(0, 1), pl.ds(0, _TAIL)], tout_sem).start()

    for c in range(_NCH):
        if c >= _NBUF:
            _out_copy(out_hbm, stage, out_sems, base, c - _NBUF).wait()
            _in_copy(x_hbm, stage, in_sems, base, c).start()
        _in_copy(x_hbm, stage, in_sems, base, c).wait()
        _out_copy(out_hbm, stage, out_sems, base, c).start()

    for c in range(_NCH - _NBUF, _NCH):
        _out_copy(out_hbm, stage, out_sems, base, c).wait()

    @pl.when(cid == 0)
    def _():
        pltpu.make_async_copy(
            tstage, out_hbm.at[pl.ds(0, 1), pl.ds(0, _TAIL)], tout_sem).wait()


def kernel(x, buffer):
    return _sc_concat(x, buffer)


# speedup vs baseline: 1.0943x; 1.0943x over previous
import functools

import jax
import jax.numpy as jnp
from jax import lax
from jax.experimental import pallas as pl
from jax.experimental.pallas import tpu as pltpu
from jax.experimental.pallas import tpu_sc as plsc

_BUFFER_SIZE = 4194304
_TAIL = 1536
_BUF_LEN = _BUFFER_SIZE + _TAIL
_NC = 2
_NS = 16
_NW = _NC * _NS
_PER_W = _BUFFER_SIZE // _NW
_CH = 32768
_NCH = _PER_W // _CH
_NBUF = 3

_MESH = plsc.VectorSubcoreMesh(core_axis_name="c", subcore_axis_name="s")


def _in_copy(x_hbm, stage, in_sems, base, c):
    return pltpu.make_async_copy(
        x_hbm.at[pl.ds(0, 1), pl.ds(base + c * _CH, _CH)],
        stage.at[pl.ds(c % _NBUF, 1), :],
        in_sems.at[c % _NBUF],
    )


def _out_copy(out_hbm, stage, out_sems, base, c):
    return pltpu.make_async_copy(
        stage.at[pl.ds(c % _NBUF, 1), :],
        out_hbm.at[pl.ds(0, 1), pl.ds(_TAIL + base + c * _CH, _CH)],
        out_sems.at[c % _NBUF],
    )


@functools.partial(
    pl.kernel,
    out_type=jax.ShapeDtypeStruct((1, _BUF_LEN), jnp.float32),
    mesh=_MESH,
    scratch_types=[
        pltpu.VMEM((_NBUF + 1, _CH), jnp.float32),
        pltpu.SemaphoreType.DMA((_NBUF,)),
        pltpu.SemaphoreType.DMA((_NBUF,)),
        pltpu.SemaphoreType.DMA,
        pltpu.SemaphoreType.DMA,
    ],
)
def _sc_concat(x_hbm, buf_hbm, out_hbm, stage, in_sems, out_sems,
               tin_sem, tout_sem):
    wid = lax.axis_index("s") * _NC + lax.axis_index("c")
    base = wid * _PER_W

    for b in range(_NBUF):
        _in_copy(x_hbm, stage, in_sems, base, b).start()

    @pl.when(wid == 0)
    def _():
        tslot = stage.at[pl.ds(_NBUF, 1), pl.ds(0, _TAIL)]
        t_in = pltpu.make_async_copy(
            buf_hbm.at[pl.ds(0, 1), pl.ds(_BUFFER_SIZE, _TAIL)], tslot, tin_sem)
        t_in.start()
        t_in.wait()
        pltpu.make_async_copy(
            tslot, out_hbm.at[pl.ds(0, 1), pl.ds(0, _TAIL)], tout_sem).start()

    for c in range(_NCH):
        if c >= _NBUF:
            _out_copy(out_hbm, stage, out_sems, base, c - _NBUF).wait()
            _in_copy(x_hbm, stage, in_sems, base, c).start()
        _in_copy(x_hbm, stage, in_sems, base, c).wait()
        _out_copy(out_hbm, stage, out_sems, base, c).start()

    for c in range(_NCH - _NBUF, _NCH):
        _out_copy(out_hbm, stage, out_sems, base, c).wait()

    @pl.when(wid == 0)
    def _():
        pltpu.make_async_copy(
            stage.at[pl.ds(_NBUF, 1), pl.ds(0, _TAIL)],
            out_hbm.at[pl.ds(0, 1), pl.ds(0, _TAIL)], tout_sem).wait()


def kernel(x, buffer):
    return _sc_concat(x, buffer)


# speedup vs baseline: 1.1509x; 1.0517x over previous
import functools

import jax
import jax.numpy as jnp
from jax import lax
from jax.experimental import pallas as pl
from jax.experimental.pallas import tpu as pltpu
from jax.experimental.pallas import tpu_sc as plsc

_BUFFER_SIZE = 4194304
_TAIL = 1536
_BUF_LEN = _BUFFER_SIZE + _TAIL
_NC = 2
_NS = 16
_NW = _NC * _NS
_PER_W = _BUFFER_SIZE // _NW
_CH = 32768
_NCH = _PER_W // _CH

_MESH = plsc.VectorSubcoreMesh(core_axis_name="c", subcore_axis_name="s")


def _in_copy(x_hbm, stage, in_sems, base, c):
    return pltpu.make_async_copy(
        x_hbm.at[pl.ds(0, 1), pl.ds(base + c * _CH, _CH)],
        stage.at[pl.ds(c, 1), :],
        in_sems.at[c],
    )


def _out_copy(out_hbm, stage, out_sems, base, c):
    return pltpu.make_async_copy(
        stage.at[pl.ds(c, 1), :],
        out_hbm.at[pl.ds(0, 1), pl.ds(_TAIL + base + c * _CH, _CH)],
        out_sems.at[c],
    )


@functools.partial(
    pl.kernel,
    out_type=jax.ShapeDtypeStruct((1, _BUF_LEN), jnp.float32),
    mesh=_MESH,
    scratch_types=[
        pltpu.VMEM((_NCH, _CH), jnp.float32),
        pltpu.SemaphoreType.DMA((_NCH,)),
        pltpu.SemaphoreType.DMA((_NCH,)),
        pltpu.SemaphoreType.DMA,
        pltpu.SemaphoreType.DMA,
    ],
)
def _sc_concat(x_hbm, buf_hbm, out_hbm, stage, in_sems, out_sems,
               tin_sem, tout_sem):
    wid = lax.axis_index("s") * _NC + lax.axis_index("c")
    base = wid * _PER_W

    for c in range(_NCH):
        _in_copy(x_hbm, stage, in_sems, base, c).start()

    for c in range(_NCH):
        _in_copy(x_hbm, stage, in_sems, base, c).wait()
        _out_copy(out_hbm, stage, out_sems, base, c).start()

    _out_copy(out_hbm, stage, out_sems, base, 0).wait()

    @pl.when(wid == 0)
    def _():
        tslot = stage.at[pl.ds(0, 1), pl.ds(0, _TAIL)]
        t_in = pltpu.make_async_copy(
            buf_hbm.at[pl.ds(0, 1), pl.ds(_BUFFER_SIZE, _TAIL)], tslot,
            tin_sem)
        t_in.start()
        t_in.wait()
        pltpu.make_async_copy(
            tslot, out_hbm.at[pl.ds(0, 1), pl.ds(0, _TAIL)], tout_sem).start()

    for c in range(1, _NCH):
        _out_copy(out_hbm, stage, out_sems, base, c).wait()

    @pl.when(wid == 0)
    def _():
        pltpu.make_async_copy(
            stage.at[pl.ds(0, 1), pl.ds(0, _TAIL)],
            out_hbm.at[pl.ds(0, 1), pl.ds(0, _TAIL)], tout_sem).wait()


def kernel(x, buffer):
    return _sc_concat(x, buffer)
